# baseline (speedup 1.0000x reference)
import dataclasses
import functools

import jax
import jax.numpy as jnp
from jax import lax
from jax.experimental import pallas as pl
from jax.experimental.pallas import tpu as pltpu
from jax.experimental.pallas import tpu_sc as plsc

NC = 2
NS = 16
L = 16
NW = NC * NS

BATCH = 16384
D = 32
G = 128
BPW = BATCH // NW
CH = 256
NCHUNK = BPW // CH


def _mf_body(uidx_hbm, iidx_hbm, wg_hbm, hg_hbm, out_hbm,
             uidx_v, iidx_v, ugidx_v, igidx_v, u_v, v_v, out_v, sem_u, sem_v):
    wid = lax.axis_index("c") * NS + lax.axis_index("s")
    base = wid * BPW

    pltpu.sync_copy(uidx_hbm.at[pl.ds(base, BPW)], uidx_v)
    pltpu.sync_copy(iidx_hbm.at[pl.ds(base, BPW)], iidx_v)

    @pl.loop(0, BPW, step=L)
    def _(i):
        ugidx_v[pl.ds(i, L)] = jax.lax.shift_right_logical(uidx_v[pl.ds(i, L)], 2)
        igidx_v[pl.ds(i, L)] = jax.lax.shift_right_logical(iidx_v[pl.ds(i, L)], 2)

    iota = lax.iota(jnp.int32, L)

    @pl.loop(0, NCHUNK)
    def _(c):
        co = c * CH
        cu = pltpu.async_copy(wg_hbm.at[ugidx_v.at[pl.ds(co, CH)]], u_v, sem_u)
        cv = pltpu.async_copy(hg_hbm.at[igidx_v.at[pl.ds(co, CH)]], v_v, sem_v)
        cu.wait()
        cv.wait()

        @pl.loop(0, CH, step=L)
        def _(b):
            urem = jnp.bitwise_and(uidx_v[pl.ds(co + b, L)], 3) * D
            irem = jnp.bitwise_and(iidx_v[pl.ds(co + b, L)], 3) * D
            rows = b + iota
            acc = jnp.zeros((L,), jnp.float32)
            for j in range(D):
                cols = jnp.bitwise_and(iota + j, D - 1)
                u = plsc.load_gather(u_v, [rows, urem + cols])
                v = plsc.load_gather(v_v, [rows, irem + cols])
                acc = acc + u * v
            out_v[pl.ds(co + b, L)] = 1.0 / (1.0 + jnp.exp(-acc))

    pltpu.sync_copy(out_v, out_hbm.at[pl.ds(base, BPW)])


def _compiler_params():
    cp = pltpu.CompilerParams()
    fields = pltpu.CompilerParams.__dataclass_fields__
    if "needs_layout_passes" in fields:
        cp = dataclasses.replace(cp, needs_layout_passes=False)
    return cp


def kernel(user_idx, item_idx, W, H):
    Wg = jnp.reshape(W, (W.shape[0] * D // G, G))
    Hg = jnp.reshape(H, (H.shape[0] * D // G, G))
    mesh = plsc.VectorSubcoreMesh(core_axis_name="c", subcore_axis_name="s")
    mf = functools.partial(
        pl.kernel,
        out_type=jax.ShapeDtypeStruct((BATCH,), jnp.float32),
        mesh=mesh,
        scratch_types=[
            pltpu.VMEM((BPW,), jnp.int32),
            pltpu.VMEM((BPW,), jnp.int32),
            pltpu.VMEM((BPW,), jnp.int32),
            pltpu.VMEM((BPW,), jnp.int32),
            pltpu.VMEM((CH, G), jnp.float32),
            pltpu.VMEM((CH, G), jnp.float32),
            pltpu.VMEM((BPW,), jnp.float32),
            pltpu.SemaphoreType.DMA,
            pltpu.SemaphoreType.DMA,
        ],
        compiler_params=_compiler_params(),
    )(_mf_body)
    return mf(user_idx.astype(jnp.int32), item_idx.astype(jnp.int32), Wg, Hg)


# speedup vs baseline: 1.0677x; 1.0677x over previous
import dataclasses
import functools

import jax
import jax.numpy as jnp
from jax import lax
from jax.experimental import pallas as pl
from jax.experimental.pallas import tpu as pltpu
from jax.experimental.pallas import tpu_sc as plsc

NC = 2
NS = 16
L = 16
NW = NC * NS

BATCH = 16384
D = 32
G = 128
BPW = BATCH // NW
CH = 256
NCHUNK = BPW // CH


def _mf_body(uidx_hbm, iidx_hbm, wg_hbm, hg_hbm, out_hbm,
             uidx_v, iidx_v, ugidx_v, igidx_v, u_v, v_v, out_v, sem_u, sem_v):
    wid = lax.axis_index("c") * NS + lax.axis_index("s")
    base = wid * BPW

    pltpu.sync_copy(uidx_hbm.at[pl.ds(base, BPW)], uidx_v)
    pltpu.sync_copy(iidx_hbm.at[pl.ds(base, BPW)], iidx_v)

    @pl.loop(0, BPW, step=L)
    def _(i):
        u = uidx_v[pl.ds(i, L)]
        t = iidx_v[pl.ds(i, L)]
        ugidx_v[pl.ds(i, L)] = (
            jax.lax.shift_left(jax.lax.shift_right_logical(u, 11), 9)
            + jnp.bitwise_and(u, 511))
        igidx_v[pl.ds(i, L)] = (
            jax.lax.shift_left(jax.lax.shift_right_logical(t, 11), 9)
            + jnp.bitwise_and(t, 511))

    iota = lax.iota(jnp.int32, L)

    @pl.loop(0, NCHUNK)
    def _(c):
        co = c * CH
        cu = pltpu.async_copy(wg_hbm.at[ugidx_v.at[pl.ds(co, CH)]], u_v, sem_u)
        cv = pltpu.async_copy(hg_hbm.at[igidx_v.at[pl.ds(co, CH)]], v_v, sem_v)
        cu.wait()
        cv.wait()

        @pl.loop(0, CH, step=L)
        def _(b):
            urem = jnp.bitwise_and(
                jax.lax.shift_right_logical(uidx_v[pl.ds(co + b, L)], 9), 3) * D
            irem = jnp.bitwise_and(
                jax.lax.shift_right_logical(iidx_v[pl.ds(co + b, L)], 9), 3) * D
            rows = b + iota
            acc = jnp.zeros((L,), jnp.float32)
            for j in range(D):
                cols = jnp.bitwise_and(iota + j, D - 1)
                u = plsc.load_gather(u_v, [rows, urem + cols])
                v = plsc.load_gather(v_v, [rows, irem + cols])
                acc = acc + u * v
            out_v[pl.ds(co + b, L)] = 1.0 / (1.0 + jnp.exp(-acc))

    pltpu.sync_copy(out_v, out_hbm.at[pl.ds(base, BPW)])


def _compiler_params():
    cp = pltpu.CompilerParams()
    fields = pltpu.CompilerParams.__dataclass_fields__
    if "needs_layout_passes" in fields:
        cp = dataclasses.replace(cp, needs_layout_passes=False)
    return cp


_TB = 2048
_SUB = _TB // 4


def _tp_body(wt_ref, og_ref):
    for m in range(4):
        og_ref[:, pl.ds(m * D, D)] = wt_ref[:, pl.ds(m * _SUB, _SUB)].T


def _regroup(Wt):
    n = Wt.shape[1]
    grid = (n + _TB - 1) // _TB
    return pl.pallas_call(
        _tp_body,
        out_shape=jax.ShapeDtypeStruct((grid * _SUB, G), jnp.float32),
        grid=(grid,),
        in_specs=[pl.BlockSpec((D, _TB), lambda i: (0, i))],
        out_specs=pl.BlockSpec((_SUB, G), lambda i: (i, 0)),
    )(Wt)


def kernel(user_idx, item_idx, W, H):
    Wg = _regroup(W.T)
    Hg = _regroup(H.T)
    mesh = plsc.VectorSubcoreMesh(core_axis_name="c", subcore_axis_name="s")
    mf = functools.partial(
        pl.kernel,
        out_type=jax.ShapeDtypeStruct((BATCH,), jnp.float32),
        mesh=mesh,
        scratch_types=[
            pltpu.VMEM((BPW,), jnp.int32),
            pltpu.VMEM((BPW,), jnp.int32),
            pltpu.VMEM((BPW,), jnp.int32),
            pltpu.VMEM((BPW,), jnp.int32),
            pltpu.VMEM((CH, G), jnp.float32),
            pltpu.VMEM((CH, G), jnp.float32),
            pltpu.VMEM((BPW,), jnp.float32),
            pltpu.SemaphoreType.DMA,
            pltpu.SemaphoreType.DMA,
        ],
        compiler_params=_compiler_params(),
    )(_mf_body)
    return mf(user_idx.astype(jnp.int32), item_idx.astype(jnp.int32), Wg, Hg)


# speedup vs baseline: 1.2954x; 1.2132x over previous
import dataclasses
import functools

import jax
import jax.numpy as jnp
from jax import lax
from jax.experimental import pallas as pl
from jax.experimental.pallas import tpu as pltpu
from jax.experimental.pallas import tpu_sc as plsc

NC = 2
NS = 16
L = 16
NW = NC * NS

BATCH = 16384
D = 32
G = 128
BPW = BATCH // NW
CH = 256
NCHUNK = BPW // CH
_Q = 262144


def _mf_body(uidx_hbm, iidx_hbm, wg_hbm, hg_hbm, out_hbm,
             uidx_v, iidx_v, ugidx_v, igidx_v, u_v, v_v, out_v, sem_u, sem_v):
    wid = lax.axis_index("c") * NS + lax.axis_index("s")
    base = wid * BPW

    pltpu.sync_copy(uidx_hbm.at[pl.ds(base, BPW)], uidx_v)
    pltpu.sync_copy(iidx_hbm.at[pl.ds(base, BPW)], iidx_v)

    @pl.loop(0, BPW, step=L)
    def _(i):
        u = uidx_v[pl.ds(i, L)]
        t = iidx_v[pl.ds(i, L)]
        ugidx_v[pl.ds(i, L)] = jnp.bitwise_and(u, _Q - 1)
        igidx_v[pl.ds(i, L)] = jnp.bitwise_and(t, _Q - 1)

    iota = lax.iota(jnp.int32, L)

    @pl.loop(0, NCHUNK)
    def _(c):
        co = c * CH
        cu = pltpu.async_copy(wg_hbm.at[ugidx_v.at[pl.ds(co, CH)]], u_v, sem_u)
        cv = pltpu.async_copy(hg_hbm.at[igidx_v.at[pl.ds(co, CH)]], v_v, sem_v)
        cu.wait()
        cv.wait()

        @pl.loop(0, CH, step=L)
        def _(b):
            urem = jax.lax.shift_right_logical(uidx_v[pl.ds(co + b, L)], 18) * D
            irem = jax.lax.shift_right_logical(iidx_v[pl.ds(co + b, L)], 18) * D
            rows = b + iota
            acc = jnp.zeros((L,), jnp.float32)
            for j in range(D):
                cols = jnp.bitwise_and(iota + j, D - 1)
                u = plsc.load_gather(u_v, [rows, urem + cols])
                v = plsc.load_gather(v_v, [rows, irem + cols])
                acc = acc + u * v
            out_v[pl.ds(co + b, L)] = 1.0 / (1.0 + jnp.exp(-acc))

    pltpu.sync_copy(out_v, out_hbm.at[pl.ds(base, BPW)])


def _compiler_params():
    cp = pltpu.CompilerParams()
    fields = pltpu.CompilerParams.__dataclass_fields__
    if "needs_layout_passes" in fields:
        cp = dataclasses.replace(cp, needs_layout_passes=False)
    return cp


_SUB = 512
_TGRID = _Q // _SUB


def _tp_body(w0_ref, w1_ref, w2_ref, w3_ref, og_ref):
    y = jnp.concatenate(
        [w0_ref[...], w1_ref[...], w2_ref[...], w3_ref[...]], axis=0)
    og_ref[...] = y.T


def _regroup(Wt):
    last_blk = (Wt.shape[1] - 1) // _SUB

    def spec(m):
        return pl.BlockSpec(
            (D, _SUB), lambda i, m=m: (0, jnp.minimum(_TGRID * m + i, last_blk)))
    return pl.pallas_call(
        _tp_body,
        out_shape=jax.ShapeDtypeStruct((_Q, G), jnp.float32),
        grid=(_TGRID,),
        in_specs=[spec(0), spec(1), spec(2), spec(3)],
        out_specs=pl.BlockSpec((_SUB, G), lambda i: (i, 0)),
    )(Wt, Wt, Wt, Wt)


def kernel(user_idx, item_idx, W, H):
    Wg = _regroup(W.T)
    Hg = _regroup(H.T)
    mesh = plsc.VectorSubcoreMesh(core_axis_name="c", subcore_axis_name="s")
    mf = functools.partial(
        pl.kernel,
        out_type=jax.ShapeDtypeStruct((BATCH,), jnp.float32),
        mesh=mesh,
        scratch_types=[
            pltpu.VMEM((BPW,), jnp.int32),
            pltpu.VMEM((BPW,), jnp.int32),
            pltpu.VMEM((BPW,), jnp.int32),
            pltpu.VMEM((BPW,), jnp.int32),
            pltpu.VMEM((CH, G), jnp.float32),
            pltpu.VMEM((CH, G), jnp.float32),
            pltpu.VMEM((BPW,), jnp.float32),
            pltpu.SemaphoreType.DMA,
            pltpu.SemaphoreType.DMA,
        ],
        compiler_params=_compiler_params(),
    )(_mf_body)
    return mf(user_idx.astype(jnp.int32), item_idx.astype(jnp.int32), Wg, Hg)


# speedup vs baseline: 2.8607x; 2.2084x over previous
import dataclasses
import functools

import jax
import jax.numpy as jnp
from jax import lax
from jax.experimental import pallas as pl
from jax.experimental.pallas import tpu as pltpu
from jax.experimental.pallas import tpu_sc as plsc

NC = 2
NS = 16
L = 16
NW = NC * NS

BATCH = 16384
D = 32
G = 128
BPW = BATCH // NW
CH = 256
NCHUNK = BPW // CH
_Q = 262144


def _mf_body(uidx_hbm, iidx_hbm, wg_hbm, hg_hbm, out_hbm,
             uidx_v, iidx_v, ugidx_v, igidx_v, u_v, v_v, out_v, sem_u, sem_v):
    wid = lax.axis_index("c") * NS + lax.axis_index("s")
    base = wid * BPW

    pltpu.sync_copy(uidx_hbm.at[pl.ds(base, BPW)], uidx_v)
    pltpu.sync_copy(iidx_hbm.at[pl.ds(base, BPW)], iidx_v)

    @pl.loop(0, BPW, step=L)
    def _(i):
        u = uidx_v[pl.ds(i, L)]
        t = iidx_v[pl.ds(i, L)]
        ugidx_v[pl.ds(i, L)] = jnp.bitwise_and(u, _Q - 1)
        igidx_v[pl.ds(i, L)] = jnp.bitwise_and(t, _Q - 1)

    iota = lax.iota(jnp.int32, L)

    @pl.loop(0, NCHUNK)
    def _(c):
        co = c * CH
        cu = pltpu.async_copy(wg_hbm.at[ugidx_v.at[pl.ds(co, CH)]], u_v, sem_u)
        cv = pltpu.async_copy(hg_hbm.at[igidx_v.at[pl.ds(co, CH)]], v_v, sem_v)
        cu.wait()
        cv.wait()

        @pl.loop(0, CH, step=L)
        def _(b):
            urem = jax.lax.shift_right_logical(uidx_v[pl.ds(co + b, L)], 18) * D
            irem = jax.lax.shift_right_logical(iidx_v[pl.ds(co + b, L)], 18) * D
            rows = b + iota
            acc = jnp.zeros((L,), jnp.float32)
            for j in range(D):
                cols = jnp.bitwise_and(iota + j, D - 1)
                u = plsc.load_gather(u_v, [rows, urem + cols])
                v = plsc.load_gather(v_v, [rows, irem + cols])
                acc = acc + u * v
            out_v[pl.ds(co + b, L)] = 1.0 / (1.0 + jnp.exp(-acc))

    pltpu.sync_copy(out_v, out_hbm.at[pl.ds(base, BPW)])


def _compiler_params():
    cp = pltpu.CompilerParams()
    fields = pltpu.CompilerParams.__dataclass_fields__
    if "needs_layout_passes" in fields:
        cp = dataclasses.replace(cp, needs_layout_passes=False)
    return cp


_SUB = 2048
_TGRID = _Q // _SUB


def _tp_body(w0_ref, w1_ref, w2_ref, w3_ref, og_ref):
    y = jnp.concatenate(
        [w0_ref[...], w1_ref[...], w2_ref[...], w3_ref[...]], axis=0)
    og_ref[...] = y.T


def _regroup(Wt):
    last_blk = (Wt.shape[1] - 1) // _SUB

    def spec(m):
        return pl.BlockSpec(
            (D, _SUB), lambda i, m=m: (0, jnp.minimum(_TGRID * m + i, last_blk)))
    return pl.pallas_call(
        _tp_body,
        out_shape=jax.ShapeDtypeStruct((_Q, G), jnp.float32),
        grid=(_TGRID,),
        in_specs=[spec(0), spec(1), spec(2), spec(3)],
        out_specs=pl.BlockSpec((_SUB, G), lambda i: (i, 0)),
    )(Wt, Wt, Wt, Wt)


def kernel(user_idx, item_idx, W, H):
    Wg = _regroup(W.T)
    Hg = _regroup(H.T)
    mesh = plsc.VectorSubcoreMesh(core_axis_name="c", subcore_axis_name="s")
    mf = functools.partial(
        pl.kernel,
        out_type=jax.ShapeDtypeStruct((BATCH,), jnp.float32),
        mesh=mesh,
        scratch_types=[
            pltpu.VMEM((BPW,), jnp.int32),
            pltpu.VMEM((BPW,), jnp.int32),
            pltpu.VMEM((BPW,), jnp.int32),
            pltpu.VMEM((BPW,), jnp.int32),
            pltpu.VMEM((CH, G), jnp.float32),
            pltpu.VMEM((CH, G), jnp.float32),
            pltpu.VMEM((BPW,), jnp.float32),
            pltpu.SemaphoreType.DMA,
            pltpu.SemaphoreType.DMA,
        ],
        compiler_params=_compiler_params(),
    )(_mf_body)
    return mf(user_idx.astype(jnp.int32), item_idx.astype(jnp.int32), Wg, Hg)


# speedup vs baseline: 3.9008x; 1.3636x over previous
import dataclasses
import functools

import jax
import jax.numpy as jnp
from jax import lax
from jax.experimental import pallas as pl
from jax.experimental.pallas import tpu as pltpu
from jax.experimental.pallas import tpu_sc as plsc

NC = 2
NS = 16
L = 16
NW = NC * NS

BATCH = 16384
D = 32
G = 128
BPW = BATCH // NW
CH = 256
NCHUNK = BPW // CH
_Q = 262144


def _mf_body(uidx_hbm, iidx_hbm, wg_hbm, hg_hbm, out_hbm,
             uidx_v, iidx_v, ugidx_v, igidx_v, u_v, v_v, out_v, sem_u, sem_v):
    wid = lax.axis_index("c") * NS + lax.axis_index("s")
    base = wid * BPW

    pltpu.sync_copy(uidx_hbm.at[pl.ds(base, BPW)], uidx_v)
    pltpu.sync_copy(iidx_hbm.at[pl.ds(base, BPW)], iidx_v)

    @pl.loop(0, BPW, step=L)
    def _(i):
        u = uidx_v[pl.ds(i, L)]
        t = iidx_v[pl.ds(i, L)]
        ugidx_v[pl.ds(i, L)] = jnp.bitwise_and(u, _Q - 1)
        igidx_v[pl.ds(i, L)] = jnp.bitwise_and(t, _Q - 1)

    iota = lax.iota(jnp.int32, L)

    @pl.loop(0, NCHUNK)
    def _(c):
        co = c * CH
        cu = pltpu.async_copy(wg_hbm.at[ugidx_v.at[pl.ds(co, CH)]], u_v, sem_u)
        cv = pltpu.async_copy(hg_hbm.at[igidx_v.at[pl.ds(co, CH)]], v_v, sem_v)
        cu.wait()
        cv.wait()

        @pl.loop(0, CH, step=L)
        def _(b):
            urem = jax.lax.shift_right_logical(uidx_v[pl.ds(co + b, L)], 18) * D
            irem = jax.lax.shift_right_logical(iidx_v[pl.ds(co + b, L)], 18) * D
            rows = b + iota
            acc = jnp.zeros((L,), jnp.float32)
            for j in range(D):
                cols = jnp.bitwise_and(iota + j, D - 1)
                u = plsc.load_gather(u_v, [rows, urem + cols])
                v = plsc.load_gather(v_v, [rows, irem + cols])
                acc = acc + u * v
            out_v[pl.ds(co + b, L)] = 1.0 / (1.0 + jnp.exp(-acc))

    pltpu.sync_copy(out_v, out_hbm.at[pl.ds(base, BPW)])


def _compiler_params():
    cp = pltpu.CompilerParams()
    fields = pltpu.CompilerParams.__dataclass_fields__
    if "needs_layout_passes" in fields:
        cp = dataclasses.replace(cp, needs_layout_passes=False)
    return cp


_SUB = 4096
_TGRID = _Q // _SUB


def _tp_body(w0_ref, w1_ref, w2_ref, w3_ref, og_ref):
    y = jnp.concatenate(
        [w0_ref[...], w1_ref[...], w2_ref[...], w3_ref[...]], axis=0)
    og_ref[...] = y.T


def _regroup(Wt):
    last_blk = (Wt.shape[1] - 1) // _SUB

    def spec(m):
        return pl.BlockSpec(
            (D, _SUB), lambda i, m=m: (0, jnp.minimum(_TGRID * m + i, last_blk)))
    return pl.pallas_call(
        _tp_body,
        out_shape=jax.ShapeDtypeStruct((_Q, G), jnp.float32),
        grid=(_TGRID,),
        in_specs=[spec(0), spec(1), spec(2), spec(3)],
        out_specs=pl.BlockSpec((_SUB, G), lambda i: (i, 0)),
    )(Wt, Wt, Wt, Wt)


def kernel(user_idx, item_idx, W, H):
    Wg = _regroup(W.T)
    Hg = _regroup(H.T)
    mesh = plsc.VectorSubcoreMesh(core_axis_name="c", subcore_axis_name="s")
    mf = functools.partial(
        pl.kernel,
        out_type=jax.ShapeDtypeStruct((BATCH,), jnp.float32),
        mesh=mesh,
        scratch_types=[
            pltpu.VMEM((BPW,), jnp.int32),
            pltpu.VMEM((BPW,), jnp.int32),
            pltpu.VMEM((BPW,), jnp.int32),
            pltpu.VMEM((BPW,), jnp.int32),
            pltpu.VMEM((CH, G), jnp.float32),
            pltpu.VMEM((CH, G), jnp.float32),
            pltpu.VMEM((BPW,), jnp.float32),
            pltpu.SemaphoreType.DMA,
            pltpu.SemaphoreType.DMA,
        ],
        compiler_params=_compiler_params(),
    )(_mf_body)
    return mf(user_idx.astype(jnp.int32), item_idx.astype(jnp.int32), Wg, Hg)


# speedup vs baseline: 4.4665x; 1.1450x over previous
import dataclasses
import functools

import jax
import jax.numpy as jnp
from jax import lax
from jax.experimental import pallas as pl
from jax.experimental.pallas import tpu as pltpu
from jax.experimental.pallas import tpu_sc as plsc

NC = 2
NS = 16
L = 16
NW = NC * NS

BATCH = 16384
D = 32
G = 128
BPW = BATCH // NW
CH = 256
NCHUNK = BPW // CH
_Q = 262144


def _mf_body(uidx_hbm, iidx_hbm, wg_hbm, hg_hbm, out_hbm,
             uidx_v, iidx_v, ugidx_v, igidx_v, u_v, v_v, out_v, sem_u, sem_v):
    wid = lax.axis_index("c") * NS + lax.axis_index("s")
    base = wid * BPW

    pltpu.sync_copy(uidx_hbm.at[pl.ds(base, BPW)], uidx_v)
    pltpu.sync_copy(iidx_hbm.at[pl.ds(base, BPW)], iidx_v)

    @pl.loop(0, BPW, step=L)
    def _(i):
        u = uidx_v[pl.ds(i, L)]
        t = iidx_v[pl.ds(i, L)]
        ugidx_v[pl.ds(i, L)] = jnp.bitwise_and(u, _Q - 1)
        igidx_v[pl.ds(i, L)] = jnp.bitwise_and(t, _Q - 1)

    iota = lax.iota(jnp.int32, L)

    @pl.loop(0, NCHUNK)
    def _(c):
        co = c * CH
        cu = pltpu.async_copy(wg_hbm.at[ugidx_v.at[pl.ds(co, CH)]], u_v, sem_u)
        cv = pltpu.async_copy(hg_hbm.at[igidx_v.at[pl.ds(co, CH)]], v_v, sem_v)
        cu.wait()
        cv.wait()

        @pl.loop(0, CH, step=L)
        def _(b):
            urem = jax.lax.shift_right_logical(uidx_v[pl.ds(co + b, L)], 18) * D
            irem = jax.lax.shift_right_logical(iidx_v[pl.ds(co + b, L)], 18) * D
            rows = b + iota
            acc = jnp.zeros((L,), jnp.float32)
            for j in range(D):
                cols = jnp.bitwise_and(iota + j, D - 1)
                u = plsc.load_gather(u_v, [rows, urem + cols])
                v = plsc.load_gather(v_v, [rows, irem + cols])
                acc = acc + u * v
            out_v[pl.ds(co + b, L)] = 1.0 / (1.0 + jnp.exp(-acc))

    pltpu.sync_copy(out_v, out_hbm.at[pl.ds(base, BPW)])


def _compiler_params():
    cp = pltpu.CompilerParams()
    fields = pltpu.CompilerParams.__dataclass_fields__
    if "needs_layout_passes" in fields:
        cp = dataclasses.replace(cp, needs_layout_passes=False)
    return cp


_SUB = 8192
_TGRID = _Q // _SUB


def _tp_body(w0_ref, w1_ref, w2_ref, w3_ref, og_ref):
    y = jnp.concatenate(
        [w0_ref[...], w1_ref[...], w2_ref[...], w3_ref[...]], axis=0)
    og_ref[...] = y.T


def _regroup(Wt):
    last_blk = (Wt.shape[1] - 1) // _SUB

    def spec(m):
        return pl.BlockSpec(
            (D, _SUB), lambda i, m=m: (0, jnp.minimum(_TGRID * m + i, last_blk)))
    return pl.pallas_call(
        _tp_body,
        out_shape=jax.ShapeDtypeStruct((_Q, G), jnp.float32),
        grid=(_TGRID,),
        in_specs=[spec(0), spec(1), spec(2), spec(3)],
        out_specs=pl.BlockSpec((_SUB, G), lambda i: (i, 0)),
    )(Wt, Wt, Wt, Wt)


def kernel(user_idx, item_idx, W, H):
    Wg = _regroup(W.T)
    Hg = _regroup(H.T)
    mesh = plsc.VectorSubcoreMesh(core_axis_name="c", subcore_axis_name="s")
    mf = functools.partial(
        pl.kernel,
        out_type=jax.ShapeDtypeStruct((BATCH,), jnp.float32),
        mesh=mesh,
        scratch_types=[
            pltpu.VMEM((BPW,), jnp.int32),
            pltpu.VMEM((BPW,), jnp.int32),
            pltpu.VMEM((BPW,), jnp.int32),
            pltpu.VMEM((BPW,), jnp.int32),
            pltpu.VMEM((CH, G), jnp.float32),
            pltpu.VMEM((CH, G), jnp.float32),
            pltpu.VMEM((BPW,), jnp.float32),
            pltpu.SemaphoreType.DMA,
            pltpu.SemaphoreType.DMA,
        ],
        compiler_params=_compiler_params(),
    )(_mf_body)
    return mf(user_idx.astype(jnp.int32), item_idx.astype(jnp.int32), Wg, Hg)


# speedup vs baseline: 4.5951x; 1.0288x over previous
import dataclasses
import functools

import jax
import jax.numpy as jnp
from jax import lax
from jax.experimental import pallas as pl
from jax.experimental.pallas import tpu as pltpu
from jax.experimental.pallas import tpu_sc as plsc

NC = 2
NS = 16
L = 16
NW = NC * NS

BATCH = 16384
D = 32
G = 128
BPW = BATCH // NW
CH = 256
NCHUNK = BPW // CH
_Q = 262144


def _mf_body(uidx_hbm, iidx_hbm, wg_hbm, hg_hbm, out_hbm,
             uidx_v, iidx_v, ugidx_v, igidx_v, u_v, v_v, out_v, sem_u, sem_v):
    wid = lax.axis_index("c") * NS + lax.axis_index("s")
    base = wid * BPW

    pltpu.sync_copy(uidx_hbm.at[pl.ds(base, BPW)], uidx_v)
    pltpu.sync_copy(iidx_hbm.at[pl.ds(base, BPW)], iidx_v)

    @pl.loop(0, BPW, step=L)
    def _(i):
        u = uidx_v[pl.ds(i, L)]
        t = iidx_v[pl.ds(i, L)]
        ugidx_v[pl.ds(i, L)] = jnp.bitwise_and(u, _Q - 1)
        igidx_v[pl.ds(i, L)] = jnp.bitwise_and(t, _Q - 1)

    iota = lax.iota(jnp.int32, L)

    @pl.loop(0, NCHUNK)
    def _(c):
        co = c * CH
        cu = pltpu.async_copy(wg_hbm.at[ugidx_v.at[pl.ds(co, CH)]], u_v, sem_u)
        cv = pltpu.async_copy(hg_hbm.at[igidx_v.at[pl.ds(co, CH)]], v_v, sem_v)
        cu.wait()
        cv.wait()

        @pl.loop(0, CH, step=L)
        def _(b):
            urem = jax.lax.shift_right_logical(uidx_v[pl.ds(co + b, L)], 18) * D
            irem = jax.lax.shift_right_logical(iidx_v[pl.ds(co + b, L)], 18) * D
            rows = b + iota
            acc = jnp.zeros((L,), jnp.float32)
            for j in range(D):
                cols = jnp.bitwise_and(iota + j, D - 1)
                u = plsc.load_gather(u_v, [rows, urem + cols])
                v = plsc.load_gather(v_v, [rows, irem + cols])
                acc = acc + u * v
            out_v[pl.ds(co + b, L)] = 1.0 / (1.0 + jnp.exp(-acc))

    pltpu.sync_copy(out_v, out_hbm.at[pl.ds(base, BPW)])


def _compiler_params():
    cp = pltpu.CompilerParams()
    fields = pltpu.CompilerParams.__dataclass_fields__
    if "needs_layout_passes" in fields:
        cp = dataclasses.replace(cp, needs_layout_passes=False)
    return cp


_SUB = 16384
_TGRID = _Q // _SUB


def _tp_body(w0_ref, w1_ref, w2_ref, w3_ref, og_ref):
    y = jnp.concatenate(
        [w0_ref[...], w1_ref[...], w2_ref[...], w3_ref[...]], axis=0)
    og_ref[...] = y.T


def _regroup(Wt):
    last_blk = (Wt.shape[1] - 1) // _SUB

    def spec(m):
        return pl.BlockSpec(
            (D, _SUB), lambda i, m=m: (0, jnp.minimum(_TGRID * m + i, last_blk)))
    return pl.pallas_call(
        _tp_body,
        out_shape=jax.ShapeDtypeStruct((_Q, G), jnp.float32),
        grid=(_TGRID,),
        in_specs=[spec(0), spec(1), spec(2), spec(3)],
        out_specs=pl.BlockSpec((_SUB, G), lambda i: (i, 0)),
    )(Wt, Wt, Wt, Wt)


def kernel(user_idx, item_idx, W, H):
    Wg = _regroup(W.T)
    Hg = _regroup(H.T)
    mesh = plsc.VectorSubcoreMesh(core_axis_name="c", subcore_axis_name="s")
    mf = functools.partial(
        pl.kernel,
        out_type=jax.ShapeDtypeStruct((BATCH,), jnp.float32),
        mesh=mesh,
        scratch_types=[
            pltpu.VMEM((BPW,), jnp.int32),
            pltpu.VMEM((BPW,), jnp.int32),
            pltpu.VMEM((BPW,), jnp.int32),
            pltpu.VMEM((BPW,), jnp.int32),
            pltpu.VMEM((CH, G), jnp.float32),
            pltpu.VMEM((CH, G), jnp.float32),
            pltpu.VMEM((BPW,), jnp.float32),
            pltpu.SemaphoreType.DMA,
            pltpu.SemaphoreType.DMA,
        ],
        compiler_params=_compiler_params(),
    )(_mf_body)
    return mf(user_idx.astype(jnp.int32), item_idx.astype(jnp.int32), Wg, Hg)
